# Initial kernel scaffold; baseline (speedup 1.0000x reference)
import functools

import jax
import jax.numpy as jnp
from jax import lax
from jax.experimental import pallas as pl
from jax.experimental.pallas import tpu as pltpu
from jax.experimental.pallas import tpu_sc as plsc

_EPS = 1e-12
_NC = 2
_NS = 16
_NW = _NC * _NS
_LANES = 16
_CHUNK = 128


def _rsqrt(x):
    i = plsc.bitcast(x, jnp.int32)
    i = 0x5F3759DF - lax.shift_right_arithmetic(i, 1)
    y = plsc.bitcast(i, jnp.float32)
    for _ in range(3):
        y = y * (1.5 - 0.5 * x * y * y)
    return y


def _make_sc_kernel(n_tokens, emb):
    per_w = n_tokens // _NW
    n_chunks = per_w // _CHUNK
    n_sub = emb // _LANES
    mesh = plsc.VectorSubcoreMesh(core_axis_name="c", subcore_axis_name="s")

    @functools.partial(
        pl.kernel,
        mesh=mesh,
        out_type=jax.ShapeDtypeStruct((n_tokens, emb), jnp.float32),
        scratch_types=[
            pltpu.VMEM((_CHUNK,), jnp.int32),
            pltpu.VMEM((_CHUNK,), jnp.int32),
            pltpu.VMEM((_CHUNK, emb), jnp.float32),
            pltpu.VMEM((_CHUNK, emb), jnp.float32),
            pltpu.VMEM((2, emb), jnp.float32),
            pltpu.SemaphoreType.DMA,
            pltpu.SemaphoreType.DMA,
        ],
    )
    def sc_kernel(ids_hbm, pids_hbm, word_hbm, ptt_hbm, gb_hbm, out_hbm,
                  idx_v, pidx_v, rows_v, prow_v, gb_v, sem_w, sem_p):
        wid = lax.axis_index("s") * _NC + lax.axis_index("c")
        base = wid * per_w
        pltpu.sync_copy(gb_hbm, gb_v)

        def chunk_body(ci, carry):
            cbase = base + ci * _CHUNK
            pltpu.sync_copy(ids_hbm.at[pl.ds(cbase, _CHUNK)], idx_v)
            pltpu.sync_copy(pids_hbm.at[pl.ds(cbase, _CHUNK)], pidx_v)
            cw = pltpu.async_copy(word_hbm.at[idx_v], rows_v, sem_w)
            cp = pltpu.async_copy(ptt_hbm.at[pidx_v], prow_v, sem_p)
            cw.wait()
            cp.wait()

            def tok_body(t, carry2):
                regs = [rows_v[t, pl.ds(k * _LANES, _LANES)]
                        + prow_v[t, pl.ds(k * _LANES, _LANES)]
                        for k in range(n_sub)]
                sv = regs[0]
                qv = regs[0] * regs[0]
                for k in range(1, n_sub):
                    sv = sv + regs[k]
                    qv = qv + regs[k] * regs[k]
                inv_n = 1.0 / emb
                mean_v = lax.broadcast(jnp.sum(sv), (_LANES,)) * inv_n
                msq_v = lax.broadcast(jnp.sum(qv), (_LANES,)) * inv_n
                var_v = msq_v - mean_v * mean_v
                inv_std = _rsqrt(var_v + _EPS)
                for k in range(n_sub):
                    g = gb_v[0, pl.ds(k * _LANES, _LANES)]
                    b = gb_v[1, pl.ds(k * _LANES, _LANES)]
                    rows_v[t, pl.ds(k * _LANES, _LANES)] = (
                        (regs[k] - mean_v) * inv_std * g + b)
                return carry2

            lax.fori_loop(0, _CHUNK, tok_body, 0)
            pltpu.sync_copy(rows_v, out_hbm.at[pl.ds(cbase, _CHUNK)])
            return carry

        lax.fori_loop(0, n_chunks, chunk_body, 0)

    return sc_kernel


@jax.jit
def kernel(input_ids, token_type_ids, word_embeddings, position_embeddings,
           token_type_embeddings, ln_gamma, ln_beta):
    bsz, seq = input_ids.shape
    vocab, emb = word_embeddings.shape
    n_tokens = bsz * seq

    ids = input_ids.astype(jnp.int32).reshape(-1)
    tv = token_type_embeddings.shape[0]
    ptt = (position_embeddings[:seq, None, :]
           + token_type_embeddings[None, :, :]).reshape(seq * tv, emb)
    pids = (jnp.arange(seq, dtype=jnp.int32)[None, :] * tv
            + token_type_ids.astype(jnp.int32)).reshape(-1)
    gb = jnp.stack([ln_gamma, ln_beta])

    sc = _make_sc_kernel(n_tokens, emb)
    out = sc(ids, pids, word_embeddings, ptt, gb)
    return out.reshape(bsz, seq, emb)


# speedup vs baseline: 2.2610x; 2.2610x over previous
import functools

import jax
import jax.numpy as jnp
from jax import lax
from jax.experimental import pallas as pl
from jax.experimental.pallas import tpu as pltpu
from jax.experimental.pallas import tpu_sc as plsc

_EPS = 1e-12
_NC = 2
_NS = 16
_NW = _NC * _NS
_LANES = 16
_CHUNK = 128


def _lane_shuffle(v, idx):
    dnums = lax.GatherDimensionNumbers(
        offset_dims=(), collapsed_slice_dims=(0,), start_index_map=(0,))
    return lax.gather(v, idx[:, None], dnums, slice_sizes=(1,),
                      mode=lax.GatherScatterMode.PROMISE_IN_BOUNDS)


def _allsum(v):
    lane = lax.iota(jnp.int32, _LANES)
    for stride in (1, 2, 4, 8):
        v = v + _lane_shuffle(v, lax.bitwise_xor(lane, stride))
    return v


def _rsqrt(x):
    i = plsc.bitcast(x, jnp.int32)
    i = 0x5F3759DF - lax.shift_right_arithmetic(i, 1)
    y = plsc.bitcast(i, jnp.float32)
    for _ in range(3):
        y = y * (1.5 - 0.5 * x * y * y)
    return y


def _make_sc_kernel(n_tokens, emb):
    per_w = n_tokens // _NW
    n_chunks = per_w // _CHUNK
    n_sub = emb // _LANES
    mesh = plsc.VectorSubcoreMesh(core_axis_name="c", subcore_axis_name="s")

    @functools.partial(
        pl.kernel,
        mesh=mesh,
        compiler_params=pltpu.CompilerParams(needs_layout_passes=False),
        out_type=jax.ShapeDtypeStruct((n_tokens, emb), jnp.float32),
        scratch_types=[
            pltpu.VMEM((_CHUNK,), jnp.int32),
            pltpu.VMEM((_CHUNK,), jnp.int32),
            pltpu.VMEM((_CHUNK, emb), jnp.float32),
            pltpu.VMEM((_CHUNK, emb), jnp.float32),
            pltpu.VMEM((2, emb), jnp.float32),
            pltpu.SemaphoreType.DMA,
            pltpu.SemaphoreType.DMA,
        ],
    )
    def sc_kernel(ids_hbm, pids_hbm, word_hbm, ptt_hbm, gb_hbm, out_hbm,
                  idx_v, pidx_v, rows_v, prow_v, gb_v, sem_w, sem_p):
        wid = lax.axis_index("s") * _NC + lax.axis_index("c")
        base = wid * per_w
        pltpu.sync_copy(gb_hbm, gb_v)

        def chunk_body(ci, carry):
            cbase = base + ci * _CHUNK
            pltpu.sync_copy(ids_hbm.at[pl.ds(cbase, _CHUNK)], idx_v)
            pltpu.sync_copy(pids_hbm.at[pl.ds(cbase, _CHUNK)], pidx_v)
            cw = pltpu.async_copy(word_hbm.at[idx_v], rows_v, sem_w)
            cp = pltpu.async_copy(ptt_hbm.at[pidx_v], prow_v, sem_p)
            cw.wait()
            cp.wait()

            def tok_body(t, carry2):
                regs = [rows_v[t, pl.ds(k * _LANES, _LANES)]
                        + prow_v[t, pl.ds(k * _LANES, _LANES)]
                        for k in range(n_sub)]
                sv = regs[0]
                qv = regs[0] * regs[0]
                for k in range(1, n_sub):
                    sv = sv + regs[k]
                    qv = qv + regs[k] * regs[k]
                inv_n = 1.0 / emb
                mean_v = _allsum(sv) * inv_n
                msq_v = _allsum(qv) * inv_n
                var_v = msq_v - mean_v * mean_v
                inv_std = _rsqrt(var_v + _EPS)
                for k in range(n_sub):
                    g = gb_v[0, pl.ds(k * _LANES, _LANES)]
                    b = gb_v[1, pl.ds(k * _LANES, _LANES)]
                    rows_v[t, pl.ds(k * _LANES, _LANES)] = (
                        (regs[k] - mean_v) * inv_std * g + b)
                return carry2

            lax.fori_loop(0, _CHUNK, tok_body, 0)
            pltpu.sync_copy(rows_v, out_hbm.at[pl.ds(cbase, _CHUNK)])
            return carry

        lax.fori_loop(0, n_chunks, chunk_body, 0)

    return sc_kernel


@jax.jit
def kernel(input_ids, token_type_ids, word_embeddings, position_embeddings,
           token_type_embeddings, ln_gamma, ln_beta):
    bsz, seq = input_ids.shape
    vocab, emb = word_embeddings.shape
    n_tokens = bsz * seq

    ids = input_ids.astype(jnp.int32).reshape(-1)
    tv = token_type_embeddings.shape[0]
    ptt = (position_embeddings[:seq, None, :]
           + token_type_embeddings[None, :, :]).reshape(seq * tv, emb)
    pids = (jnp.arange(seq, dtype=jnp.int32)[None, :] * tv
            + token_type_ids.astype(jnp.int32)).reshape(-1)
    gb = jnp.stack([ln_gamma, ln_beta])

    sc = _make_sc_kernel(n_tokens, emb)
    out = sc(ids, pids, word_embeddings, ptt, gb)
    return out.reshape(bsz, seq, emb)


# speedup vs baseline: 7.6308x; 3.3749x over previous
import functools

import jax
import jax.numpy as jnp
from jax import lax
from jax.experimental import pallas as pl
from jax.experimental.pallas import tpu as pltpu
from jax.experimental.pallas import tpu_sc as plsc

_EPS = 1e-12
_NC = 2
_NS = 16
_NW = _NC * _NS
_LANES = 16
_CHUNK = 128
_UNROLL = 2


def _lane_shuffle(v, idx):
    dnums = lax.GatherDimensionNumbers(
        offset_dims=(), collapsed_slice_dims=(0,), start_index_map=(0,))
    return lax.gather(v, idx[:, None], dnums, slice_sizes=(1,),
                      mode=lax.GatherScatterMode.PROMISE_IN_BOUNDS)


def _allsum(v):
    lane = lax.iota(jnp.int32, _LANES)
    for stride in (1, 2, 4, 8):
        v = v + _lane_shuffle(v, lax.bitwise_xor(lane, stride))
    return v


def _rsqrt(x):
    i = plsc.bitcast(x, jnp.int32)
    i = 0x5F3759DF - lax.shift_right_arithmetic(i, 1)
    y = plsc.bitcast(i, jnp.float32)
    for _ in range(2):
        y = y * (1.5 - 0.5 * x * y * y)
    return y


def _make_sc_kernel(n_tokens, emb):
    per_w = n_tokens // _NW
    n_chunks = per_w // _CHUNK
    n2 = n_chunks // 2
    n_sub = emb // _LANES
    mesh = plsc.VectorSubcoreMesh(core_axis_name="c", subcore_axis_name="s")

    @functools.partial(
        pl.kernel,
        mesh=mesh,
        compiler_params=pltpu.CompilerParams(needs_layout_passes=False),
        out_type=jax.ShapeDtypeStruct((n_tokens, emb), jnp.float32),
        scratch_types=[
            pltpu.VMEM((2, _CHUNK), jnp.int32),
            pltpu.VMEM((2, _CHUNK), jnp.int32),
            pltpu.VMEM((_CHUNK, emb), jnp.float32),
            pltpu.VMEM((_CHUNK, emb), jnp.float32),
            pltpu.VMEM((_CHUNK, emb), jnp.float32),
            pltpu.VMEM((_CHUNK, emb), jnp.float32),
            pltpu.VMEM((_CHUNK, emb), jnp.float32),
            pltpu.VMEM((_CHUNK, emb), jnp.float32),
            pltpu.VMEM((2, emb), jnp.float32),
            pltpu.SemaphoreType.DMA,
            pltpu.SemaphoreType.DMA,
            pltpu.SemaphoreType.DMA,
            pltpu.SemaphoreType.DMA,
            pltpu.SemaphoreType.DMA,
            pltpu.SemaphoreType.DMA,
        ],
    )
    def sc_kernel(pk_hbm, word_hbm, ptt_hbm, gb_hbm, out_hbm,
                  idx0, idx1, row0, row1, prw0, prw1, ob0, ob1, gb_v,
                  sw0, sw1, sp0, sp1, so0, so1):
        wid = lax.axis_index("s") * _NC + lax.axis_index("c")
        base = wid * per_w
        pltpu.sync_copy(gb_hbm, gb_v)
        gs = [gb_v[0, pl.ds(k * _LANES, _LANES)] for k in range(n_sub)]
        bs = [gb_v[1, pl.ds(k * _LANES, _LANES)] for k in range(n_sub)]

        idxs = (idx0, idx1)
        rows = (row0, row1)
        prws = (prw0, prw1)
        obs = (ob0, ob1)
        sws = (sw0, sw1)
        sps = (sp0, sp1)
        sos = (so0, so1)

        def start_gather(ci, b):
            pltpu.sync_copy(pk_hbm.at[wid, ci], idxs[b])
            pltpu.make_async_copy(
                word_hbm.at[idxs[b].at[0]], rows[b], sws[b]).start()
            pltpu.make_async_copy(
                ptt_hbm.at[idxs[b].at[1]], prws[b], sps[b]).start()

        def wait_gather(b):
            pltpu.make_async_copy(
                word_hbm.at[idxs[b].at[0]], rows[b], sws[b]).wait()
            pltpu.make_async_copy(
                ptt_hbm.at[idxs[b].at[1]], prws[b], sps[b]).wait()

        def wait_writeback(b):
            pltpu.make_async_copy(
                obs[b], out_hbm.at[pl.ds(base, _CHUNK)], sos[b]).wait()

        def compute(b):
            rv, pv, ov = rows[b], prws[b], obs[b]

            def tok_body(tt, carry):
                for j in range(_UNROLL):
                    t = tt * _UNROLL + j
                    regs = [rv[t, pl.ds(k * _LANES, _LANES)]
                            + pv[t, pl.ds(k * _LANES, _LANES)]
                            for k in range(n_sub)]
                    sv = regs[0]
                    qv = regs[0] * regs[0]
                    for k in range(1, n_sub):
                        sv = sv + regs[k]
                        qv = qv + regs[k] * regs[k]
                    inv_n = 1.0 / emb
                    mean_v = _allsum(sv) * inv_n
                    msq_v = _allsum(qv) * inv_n
                    var_v = msq_v - mean_v * mean_v
                    inv_std = _rsqrt(var_v + _EPS)
                    for k in range(n_sub):
                        ov[t, pl.ds(k * _LANES, _LANES)] = (
                            (regs[k] - mean_v) * inv_std * gs[k] + bs[k])
                return carry

            lax.fori_loop(0, _CHUNK // _UNROLL, tok_body, 0)

        def start_writeback(ci, b):
            pltpu.make_async_copy(
                obs[b], out_hbm.at[pl.ds(base + ci * _CHUNK, _CHUNK)],
                sos[b]).start()

        start_gather(0, 0)

        def loop_body(ci2, carry):
            ci_a = ci2 * 2
            ci_b = ci_a + 1
            start_gather(ci_b, 1)
            wait_gather(0)

            @pl.when(ci2 > 0)
            def _():
                wait_writeback(0)

            compute(0)
            start_writeback(ci_a, 0)

            @pl.when(ci2 < n2 - 1)
            def _():
                start_gather(ci_a + 2, 0)

            wait_gather(1)

            @pl.when(ci2 > 0)
            def _():
                wait_writeback(1)

            compute(1)
            start_writeback(ci_b, 1)
            return carry

        lax.fori_loop(0, n2, loop_body, 0)
        wait_writeback(0)
        wait_writeback(1)

    return sc_kernel


@jax.jit
def kernel(input_ids, token_type_ids, word_embeddings, position_embeddings,
           token_type_embeddings, ln_gamma, ln_beta):
    bsz, seq = input_ids.shape
    vocab, emb = word_embeddings.shape
    n_tokens = bsz * seq
    per_w = n_tokens // _NW
    n_chunks = per_w // _CHUNK

    ids = input_ids.astype(jnp.int32).reshape(-1)
    tv = token_type_embeddings.shape[0]
    ptt = (position_embeddings[:seq, None, :]
           + token_type_embeddings[None, :, :]).reshape(seq * tv, emb)
    pids = (jnp.arange(seq, dtype=jnp.int32)[None, :] * tv
            + token_type_ids.astype(jnp.int32)).reshape(-1)
    packed = jnp.stack([ids.reshape(_NW, n_chunks, _CHUNK),
                        pids.reshape(_NW, n_chunks, _CHUNK)], axis=2)
    gb = jnp.stack([ln_gamma, ln_beta])

    sc = _make_sc_kernel(n_tokens, emb)
    out = sc(packed, word_embeddings, ptt, gb)
    return out.reshape(bsz, seq, emb)


# speedup vs baseline: 7.9842x; 1.0463x over previous
import functools

import jax
import jax.numpy as jnp
from jax import lax
from jax.experimental import pallas as pl
from jax.experimental.pallas import tpu as pltpu
from jax.experimental.pallas import tpu_sc as plsc

_EPS = 1e-12
_NC = 2
_NS = 16
_NW = _NC * _NS
_LANES = 16
_CHUNK = 128
_UNROLL = 2


def _lane_shuffle(v, idx):
    dnums = lax.GatherDimensionNumbers(
        offset_dims=(), collapsed_slice_dims=(0,), start_index_map=(0,))
    return lax.gather(v, idx[:, None], dnums, slice_sizes=(1,),
                      mode=lax.GatherScatterMode.PROMISE_IN_BOUNDS)


def _allsum(v):
    lane = lax.iota(jnp.int32, _LANES)
    for stride in (1, 2, 4, 8):
        v = v + _lane_shuffle(v, lax.bitwise_xor(lane, stride))
    return v


def _rsqrt(x):
    i = plsc.bitcast(x, jnp.int32)
    i = 0x5F3759DF - lax.shift_right_arithmetic(i, 1)
    y = plsc.bitcast(i, jnp.float32)
    for _ in range(2):
        y = y * (1.5 - 0.5 * x * y * y)
    return y


def _make_sc_kernel(n_tokens, emb):
    per_w = n_tokens // _NW
    n_chunks = per_w // _CHUNK
    n2 = n_chunks // 2
    n_sub = emb // _LANES
    mesh = plsc.VectorSubcoreMesh(core_axis_name="c", subcore_axis_name="s")

    @functools.partial(
        pl.kernel,
        mesh=mesh,
        compiler_params=pltpu.CompilerParams(needs_layout_passes=False),
        out_type=jax.ShapeDtypeStruct((n_tokens, emb), jnp.float32),
        scratch_types=[
            pltpu.VMEM((per_w // _CHUNK, 2, _CHUNK), jnp.int32),
            pltpu.VMEM((_CHUNK, emb), jnp.float32),
            pltpu.VMEM((_CHUNK, emb), jnp.float32),
            pltpu.VMEM((_CHUNK, emb), jnp.float32),
            pltpu.VMEM((_CHUNK, emb), jnp.float32),
            pltpu.VMEM((_CHUNK, emb), jnp.float32),
            pltpu.VMEM((_CHUNK, emb), jnp.float32),
            pltpu.VMEM((2, emb), jnp.float32),
            pltpu.SemaphoreType.DMA,
            pltpu.SemaphoreType.DMA,
            pltpu.SemaphoreType.DMA,
            pltpu.SemaphoreType.DMA,
            pltpu.SemaphoreType.DMA,
            pltpu.SemaphoreType.DMA,
        ],
    )
    def sc_kernel(pk_hbm, word_hbm, ptt_hbm, gb_hbm, out_hbm,
                  idxall, row0, row1, prw0, prw1, ob0, ob1, gb_v,
                  sw0, sw1, sp0, sp1, so0, so1):
        wid = lax.axis_index("s") * _NC + lax.axis_index("c")
        base = wid * per_w
        pltpu.sync_copy(gb_hbm, gb_v)
        pltpu.sync_copy(pk_hbm.at[wid], idxall)
        gs = [gb_v[0, pl.ds(k * _LANES, _LANES)] for k in range(n_sub)]
        bs = [gb_v[1, pl.ds(k * _LANES, _LANES)] for k in range(n_sub)]

        rows = (row0, row1)
        prws = (prw0, prw1)
        obs = (ob0, ob1)
        sws = (sw0, sw1)
        sps = (sp0, sp1)
        sos = (so0, so1)

        def start_gather(ci, b):
            pltpu.make_async_copy(
                word_hbm.at[idxall.at[ci, 0]], rows[b], sws[b]).start()
            pltpu.make_async_copy(
                ptt_hbm.at[idxall.at[ci, 1]], prws[b], sps[b]).start()

        def wait_gather(ci, b):
            pltpu.make_async_copy(
                word_hbm.at[idxall.at[ci, 0]], rows[b], sws[b]).wait()
            pltpu.make_async_copy(
                ptt_hbm.at[idxall.at[ci, 1]], prws[b], sps[b]).wait()

        def wait_writeback(b):
            pltpu.make_async_copy(
                obs[b], out_hbm.at[pl.ds(base, _CHUNK)], sos[b]).wait()

        def compute(b):
            rv, pv, ov = rows[b], prws[b], obs[b]

            def tok_body(tt, carry):
                for j in range(_UNROLL):
                    t = tt * _UNROLL + j
                    regs = [rv[t, pl.ds(k * _LANES, _LANES)]
                            + pv[t, pl.ds(k * _LANES, _LANES)]
                            for k in range(n_sub)]
                    sv = regs[0]
                    qv = regs[0] * regs[0]
                    for k in range(1, n_sub):
                        sv = sv + regs[k]
                        qv = qv + regs[k] * regs[k]
                    inv_n = 1.0 / emb
                    mean_v = _allsum(sv) * inv_n
                    msq_v = _allsum(qv) * inv_n
                    var_v = msq_v - mean_v * mean_v
                    inv_std = _rsqrt(var_v + _EPS)
                    for k in range(n_sub):
                        ov[t, pl.ds(k * _LANES, _LANES)] = (
                            (regs[k] - mean_v) * inv_std * gs[k] + bs[k])
                return carry

            lax.fori_loop(0, _CHUNK // _UNROLL, tok_body, 0)

        def start_writeback(ci, b):
            pltpu.make_async_copy(
                obs[b], out_hbm.at[pl.ds(base + ci * _CHUNK, _CHUNK)],
                sos[b]).start()

        start_gather(0, 0)

        def loop_body(ci2, carry):
            ci_a = ci2 * 2
            ci_b = ci_a + 1
            start_gather(ci_b, 1)
            wait_gather(ci_a, 0)

            @pl.when(ci2 > 0)
            def _():
                wait_writeback(0)

            compute(0)
            start_writeback(ci_a, 0)

            @pl.when(ci2 < n2 - 1)
            def _():
                start_gather(ci_a + 2, 0)

            wait_gather(ci_b, 1)

            @pl.when(ci2 > 0)
            def _():
                wait_writeback(1)

            compute(1)
            start_writeback(ci_b, 1)
            return carry

        lax.fori_loop(0, n2, loop_body, 0)
        wait_writeback(0)
        wait_writeback(1)

    return sc_kernel


@jax.jit
def kernel(input_ids, token_type_ids, word_embeddings, position_embeddings,
           token_type_embeddings, ln_gamma, ln_beta):
    bsz, seq = input_ids.shape
    vocab, emb = word_embeddings.shape
    n_tokens = bsz * seq
    per_w = n_tokens // _NW
    n_chunks = per_w // _CHUNK

    ids = input_ids.astype(jnp.int32).reshape(-1)
    tv = token_type_embeddings.shape[0]
    ptt = (position_embeddings[:seq, None, :]
           + token_type_embeddings[None, :, :]).reshape(seq * tv, emb)
    pids = (jnp.arange(seq, dtype=jnp.int32)[None, :] * tv
            + token_type_ids.astype(jnp.int32)).reshape(-1)
    packed = jnp.stack([ids.reshape(_NW, n_chunks, _CHUNK),
                        pids.reshape(_NW, n_chunks, _CHUNK)], axis=2)
    gb = jnp.stack([ln_gamma, ln_beta])

    sc = _make_sc_kernel(n_tokens, emb)
    out = sc(packed, word_embeddings, ptt, gb)
    return out.reshape(bsz, seq, emb)
